# baseline (speedup 1.0000x reference)
import functools

import jax
import jax.numpy as jnp
from jax import lax
from jax.experimental import pallas as pl
from jax.experimental.pallas import tpu as pltpu
from jax.experimental.pallas import tpu_sc as plsc

NN = 32000
HID = 128
HEADS = 4
HD = 32
NLAYERS = 4
E = 512000
ET = E + NN
NW = 32
NPW = NN // NW
K = 128
S = 50

_IOTA16 = None


def _iota16():
    return lax.iota(jnp.int32, 16)


def _hsum_splat(v):
    iota = _iota16()
    for k in (8, 4, 2, 1):
        v = v + v.at[iota ^ k].get(mode='promise_in_bounds')
    return v


def _sread(ref, i):
    v = ref[pl.ds(i, 16)]
    return v[0]


def _worker_id():
    return lax.axis_index("c") * 16 + lax.axis_index("s")


def _edge_kernel_body(xl_h, xr_h, srcs_h, dsts_h, rp_h, att_h, out0_h,
                      out1_h, rp_v, sidx0, sidx1, didx0, didx1, xlr0, xlr1,
                      xrr0, xrr1, attv, slab, sem0, sem1):
    wid = _worker_id()
    core = wid // 16
    n0 = wid * NPW

    pltpu.sync_copy(rp_h.at[pl.ds(n0, NPW + 24)], rp_v)
    pltpu.sync_copy(att_h, attv)

    s0 = _sread(rp_v, 0)
    s1 = _sread(rp_v, NPW)
    astart = (s0 // 8) * 8
    nchunks = (s1 - astart + K - 1) // K
    sems = (sem0, sem1)
    sidxs, didxs = (sidx0, sidx1), (didx0, didx1)
    xlrs, xrrs = (xlr0, xlr1), (xrr0, xrr1)

    att_regs = [attv[pl.ds(16 * j, 16)] for j in range(8)]
    zero = jnp.zeros((16,), jnp.float32)

    def make_emit(out_h, obase):
        def emit_row(n, D, O):
            rel = n - n0
            slot = rel - (rel // S) * S
            for h in range(HEADS):
                inv = 1.0 / (D[h] + 1e-16)
                for jj in range(2):
                    j = 2 * h + jj
                    slab[pl.ds(slot * 128 + 16 * j, 16)] = O[j] * inv

            @pl.when(slot == S - 1)
            def _flush():
                pltpu.sync_copy(
                    slab,
                    out_h.at[pl.ds((n - obase - (S - 1)) * 128, S * 128)])
        return emit_row

    def make_edge_body(cbase, b, emit_row):
        xl_b, xr_b = xlrs[b], xrrs[b]

        def edge_body(e, carry):
            e_g = cbase + e
            n, end = carry[0], carry[1]
            D = list(carry[2:6])
            O = list(carry[6:14])
            bnd = e_g == end

            @pl.when(bnd)
            def _():
                emit_row(n, D, O)

            new_end = _sread(rp_v, n + 1 - n0 + 1)
            n = jnp.where(bnd, n + 1, n)
            end = jnp.where(bnd, new_end, end)
            fz = jnp.where(bnd, 0.0, 1.0)
            D = [d * fz for d in D]
            O = [o * fz for o in O]

            lrow = [xl_b[e, pl.ds(16 * j, 16)] for j in range(8)]
            rrow = [xr_b[e, pl.ds(16 * j, 16)] for j in range(8)]
            m = []
            for j in range(8):
                x = lrow[j] + rrow[j]
                m.append(jnp.maximum(x, 0.2 * x))
            for h in range(HEADS):
                u = m[2 * h] * att_regs[2 * h] + m[2 * h + 1] * att_regs[2 * h + 1]
                p = jnp.exp(_hsum_splat(u))
                D[h] = D[h] + p
                O[2 * h] = O[2 * h] + p * lrow[2 * h]
                O[2 * h + 1] = O[2 * h + 1] + p * lrow[2 * h + 1]
            return (n, end) + tuple(D) + tuple(O)
        return edge_body

    def issue(c, b):
        cbase = astart + c * K
        pltpu.sync_copy(srcs_h.at[pl.ds(cbase, K)], sidxs[b])
        pltpu.sync_copy(dsts_h.at[pl.ds(cbase, K)], didxs[b])
        pltpu.async_copy(xl_h.at[sidxs[b]], xlrs[b], sems[b])
        pltpu.async_copy(xr_h.at[didxs[b]], xrrs[b], sems[b])

    def wait(b):
        pltpu.make_async_copy(xl_h.at[sidxs[b]], xlrs[b], sems[b]).wait()
        pltpu.make_async_copy(xr_h.at[didxs[b]], xrrs[b], sems[b]).wait()

    def run(out_h, obase):
        emit_row = make_emit(out_h, obase)

        def process(c, b, carry):
            cbase = astart + c * K
            lo = jnp.maximum(s0 - cbase, 0)
            hi = jnp.minimum(K, s1 - cbase)
            return lax.fori_loop(lo, hi, make_edge_body(cbase, b, emit_row),
                                 carry)

        def pair_body(p, carry):
            ca = 2 * p
            cb = ca + 1

            @pl.when(cb < nchunks)
            def _():
                issue(cb, 1)
            wait(0)
            carry = process(ca, 0, carry)

            @pl.when(cb + 1 < nchunks)
            def _():
                issue(cb + 1, 0)

            @pl.when(cb < nchunks)
            def _():
                wait(1)
            return process(cb, 1, carry)

        issue(0, 0)
        init = (n0, _sread(rp_v, 1)) + (zero,) * 12
        npairs = (nchunks + 1) // 2
        carry = lax.fori_loop(0, npairs, pair_body, init)
        emit_row(carry[0], carry[2:6], carry[6:14])

    @pl.when(core == 0)
    def _():
        run(out0_h, 0)

    @pl.when(core == 1)
    def _():
        run(out1_h, NN // 2)


@functools.partial(jax.jit, static_argnames=())
def _noop(x):
    return x


@functools.cache
def _make_edge_call():
    mesh = plsc.VectorSubcoreMesh(core_axis_name="c", subcore_axis_name="s")
    return pl.kernel(
        _edge_kernel_body,
        out_type=[jax.ShapeDtypeStruct((NN // 2 * 128,), jnp.float32),
                  jax.ShapeDtypeStruct((NN // 2 * 128,), jnp.float32)],
        mesh=mesh,
        scratch_types=[
            pltpu.VMEM((NPW + 24,), jnp.int32),
            pltpu.VMEM((K,), jnp.int32),
            pltpu.VMEM((K,), jnp.int32),
            pltpu.VMEM((K,), jnp.int32),
            pltpu.VMEM((K,), jnp.int32),
            pltpu.VMEM((K, 128), jnp.float32),
            pltpu.VMEM((K, 128), jnp.float32),
            pltpu.VMEM((K, 128), jnp.float32),
            pltpu.VMEM((K, 128), jnp.float32),
            pltpu.VMEM((128,), jnp.float32),
            pltpu.VMEM((S * 128,), jnp.float32),
            pltpu.SemaphoreType.DMA,
            pltpu.SemaphoreType.DMA,
        ],
    )


RB = 3200


def _enc_body(p_ref, w1, b1, w2, b2, w3, b3, o_ref):
    h0 = jax.nn.silu(jnp.dot(p_ref[...], w1[...],
                             preferred_element_type=jnp.float32) + b1[...])
    h0 = jax.nn.silu(jnp.dot(h0, w2[...],
                             preferred_element_type=jnp.float32) + b2[...])
    o_ref[...] = jnp.dot(h0, w3[...],
                         preferred_element_type=jnp.float32) + b3[...]


def _enc_call(params, w):
    CB = 2560
    grid = (128000 // CB,)
    return pl.pallas_call(
        _enc_body,
        grid=grid,
        in_specs=[
            pl.BlockSpec((32, 4), lambda i: (0, 0)),
            pl.BlockSpec((4, HID), lambda i: (0, 0)),
            pl.BlockSpec((1, HID), lambda i: (0, 0)),
            pl.BlockSpec((HID, HID), lambda i: (0, 0)),
            pl.BlockSpec((1, HID), lambda i: (0, 0)),
            pl.BlockSpec((HID, CB), lambda i: (0, i)),
            pl.BlockSpec((1, CB), lambda i: (0, i)),
        ],
        out_specs=pl.BlockSpec((32, CB), lambda i: (0, i)),
        out_shape=jax.ShapeDtypeStruct((32, 128000), jnp.float32),
    )(params, w['enc_W1'], w['enc_b1'], w['enc_W2'], w['enc_b2'],
      w['enc_W3'], w['enc_b3'])


def _proj_body(h_ref, wl, bl, wr, br, xl_ref, xr_ref):
    h = h_ref[...]
    xl_ref[...] = jnp.dot(h, wl[...], preferred_element_type=jnp.float32) + bl[...]
    xr_ref[...] = jnp.dot(h, wr[...], preferred_element_type=jnp.float32) + br[...]


def _proj_call(h, wl, bl, wr, br):
    grid = (NN // RB,)
    return pl.pallas_call(
        _proj_body,
        grid=grid,
        in_specs=[
            pl.BlockSpec((RB, HID), lambda i: (i, 0)),
            pl.BlockSpec((HID, HID), lambda i: (0, 0)),
            pl.BlockSpec((1, HID), lambda i: (0, 0)),
            pl.BlockSpec((HID, HID), lambda i: (0, 0)),
            pl.BlockSpec((1, HID), lambda i: (0, 0)),
        ],
        out_specs=[
            pl.BlockSpec((RB, HID), lambda i: (i, 0)),
            pl.BlockSpec((RB, HID), lambda i: (i, 0)),
        ],
        out_shape=[
            jax.ShapeDtypeStruct((NN, HID), jnp.float32),
            jax.ShapeDtypeStruct((NN, HID), jnp.float32),
        ],
    )(h, wl, bl, wr, br)


def _post(out_raw, h_res, cb, g, be):
    out = out_raw + cb
    mu = jnp.mean(out, axis=-1, keepdims=True)
    var = jnp.mean((out - mu) ** 2, axis=-1, keepdims=True)
    out = (out - mu) / jnp.sqrt(var + 1e-5) * g + be
    out = jax.nn.silu(out)
    return out + h_res


def _postproj_body(or_ref, hr_ref, cb, g, be, wl, bl, wr, br,
                   h_ref, xl_ref, xr_ref):
    h = _post(or_ref[...], hr_ref[...], cb[...], g[...], be[...])
    h_ref[...] = h
    xl_ref[...] = jnp.dot(h, wl[...], preferred_element_type=jnp.float32) + bl[...]
    xr_ref[...] = jnp.dot(h, wr[...], preferred_element_type=jnp.float32) + br[...]


def _postproj_call(out_raw, h_res, cb, g, be, wl, bl, wr, br):
    grid = (NN // RB,)
    full = lambda r, c: pl.BlockSpec((r, c), lambda i: (0, 0))
    row = pl.BlockSpec((RB, HID), lambda i: (i, 0))
    return pl.pallas_call(
        _postproj_body,
        grid=grid,
        in_specs=[row, row, full(1, HID), full(1, HID), full(1, HID),
                  full(HID, HID), full(1, HID), full(HID, HID), full(1, HID)],
        out_specs=[row, row, row],
        out_shape=[jax.ShapeDtypeStruct((NN, HID), jnp.float32)] * 3,
    )(out_raw, h_res, cb, g, be, wl, bl, wr, br)


def _postdec_body(or_ref, hr_ref, cb, g, be, w1, b1, w2, b2, o_ref):
    h = _post(or_ref[...], hr_ref[...], cb[...], g[...], be[...])
    t = jax.nn.silu(jnp.dot(h, w1[...], preferred_element_type=jnp.float32)
                    + b1[...])
    o_ref[...] = jnp.dot(t, w2[...], preferred_element_type=jnp.float32) + b2[...]


def _postdec_call(out_raw, h_res, cb, g, be, w1, b1, w2, b2):
    grid = (NN // RB,)
    full = lambda r, c: pl.BlockSpec((r, c), lambda i: (0, 0))
    row = pl.BlockSpec((RB, HID), lambda i: (i, 0))
    return pl.pallas_call(
        _postdec_body,
        grid=grid,
        in_specs=[row, row, full(1, HID), full(1, HID), full(1, HID),
                  full(HID, HID), full(1, HID), full(HID, 1), full(1, 1)],
        out_specs=pl.BlockSpec((RB, 1), lambda i: (i, 0)),
        out_shape=jax.ShapeDtypeStruct((NN, 1), jnp.float32),
    )(out_raw, h_res, cb, g, be, w1, b1, w2, b2)


def kernel(params, weights, edge_index):
    w = weights
    loops = jnp.arange(NN, dtype=jnp.int32)
    src = jnp.concatenate([edge_index[0].astype(jnp.int32), loops])
    dst = jnp.concatenate([edge_index[1].astype(jnp.int32), loops])
    key = jnp.sort((dst << 15) | src)
    src_s = key & 0x7FFF
    dst_s = key >> 15
    rowptr = jnp.searchsorted(dst_s, jnp.arange(NN + 1, dtype=jnp.int32)
                              ).astype(jnp.int32)
    src_sp = jnp.concatenate([src_s, jnp.zeros((K,), jnp.int32)])
    dst_sp = jnp.concatenate([dst_s, jnp.zeros((K,), jnp.int32)])
    rp_pad = jnp.concatenate([rowptr, jnp.full((47,), ET, jnp.int32)])

    p4 = jnp.pad(params, ((0, 0), (0, 1)))
    w1 = jnp.pad(w['enc_W1'], ((0, 1), (0, 0)))
    hflat = _enc_call(p4, {**w, 'enc_W1': w1,
                           'enc_b1': w['enc_b1'][None, :],
                           'enc_b2': w['enc_b2'][None, :],
                           'enc_b3': w['enc_b3'][None, :]})
    h = hflat.reshape(NN, HID)

    xl, xr = _proj_call(h, w['Wl0'], w['bl0'][None, :],
                        w['Wr0'], w['br0'][None, :])
    for l in range(NLAYERS):
        att_flat = w[f'att{l}'].reshape(128)
        out0, out1 = _make_edge_call()(xl, xr, src_sp, dst_sp, rp_pad, att_flat)
        out_raw = jnp.concatenate([out0, out1]).reshape(NN, HID)
        if l < NLAYERS - 1:
            h, xl, xr = _postproj_call(
                out_raw, h, w[f'cb{l}'][None, :], w[f'g{l}'][None, :],
                w[f'be{l}'][None, :], w[f'Wl{l+1}'], w[f'bl{l+1}'][None, :],
                w[f'Wr{l+1}'], w[f'br{l+1}'][None, :])
        else:
            pred = _postdec_call(
                out_raw, h, w[f'cb{l}'][None, :], w[f'g{l}'][None, :],
                w[f'be{l}'][None, :], w['dec_W1'], w['dec_b1'][None, :],
                w['dec_W2'], w['dec_b2'][None, :])
    return pred.reshape(NN)


# speedup vs baseline: 1.3712x; 1.3712x over previous
import functools

import jax
import jax.numpy as jnp
from jax import lax
from jax.experimental import pallas as pl
from jax.experimental.pallas import tpu as pltpu
from jax.experimental.pallas import tpu_sc as plsc

NN = 32000
HID = 128
HEADS = 4
HD = 32
NLAYERS = 4
E = 512000
ET = E + NN
NW = 32
NPW = NN // NW
K = 128
S = 50

_IOTA16 = None


def _iota16():
    return lax.iota(jnp.int32, 16)


def _hsum_splat(v):
    iota = _iota16()
    for k in (8, 4, 2, 1):
        v = v + v.at[iota ^ k].get(mode='promise_in_bounds')
    return v


def _sread(ref, i):
    v = ref[pl.ds(i, 16)]
    return v[0]


def _worker_id():
    return lax.axis_index("c") * 16 + lax.axis_index("s")


def _edge_kernel_body(xl_h, xr_h, srcs_h, dsts_h, wb_h, att_h, out0_h,
                      out1_h, wbv, sidx0, sidx1, didx0, didx1, xlr0, xlr1,
                      xsl, attv, slab, sem0, sem1):
    wid = _worker_id()
    core = wid // 16
    n0 = wid * NPW

    pltpu.sync_copy(wb_h, wbv)
    pltpu.sync_copy(att_h, attv)
    pltpu.sync_copy(xr_h.at[pl.ds(n0 * 128, NPW // 2 * 128)], xsl)

    s0 = _sread(wbv, wid)
    s1 = _sread(wbv, wid + 1)
    astart = (s0 // 8) * 8
    nchunks = (s1 - astart + K - 1) // K
    sems = (sem0, sem1)
    sidxs, didxs = (sidx0, sidx1), (didx0, didx1)
    xlrs = (xlr0, xlr1)

    att_regs = [attv[pl.ds(16 * j, 16)] for j in range(8)]
    zero = jnp.zeros((16,), jnp.float32)
    iota = _iota16()

    def make_emit(out_h, obase):
        def emit_row(n, D, O):
            rel = n - n0
            slot = rel - (rel // S) * S
            for h in range(HEADS):
                inv = 1.0 / (D[h] + 1e-16)
                for jj in range(2):
                    j = 2 * h + jj
                    slab[pl.ds(slot * 128 + 16 * j, 16)] = O[j] * inv

            @pl.when(slot == S - 1)
            def _flush():
                pltpu.sync_copy(
                    slab,
                    out_h.at[pl.ds((n - obase - (S - 1)) * 128, S * 128)])
        return emit_row

    def make_edge_body(cbase, b, emit_row):
        xl_b, d_b = xlrs[b], didxs[b]

        def edge_body(e, carry):
            n = carry[0]
            D = list(carry[1:5])
            O = list(carry[5:13])

            d = d_b[pl.ds(e, 16)][0]
            bnd = d != n

            @pl.when(bnd)
            def _():
                emit_row(n, D, O)

            @pl.when(bnd & (d == n0 + NPW // 2))
            def _():
                pltpu.sync_copy(
                    xr_h.at[pl.ds((n0 + NPW // 2) * 128, NPW // 2 * 128)], xsl)

            n = jnp.where(bnd, d, n)
            fz = jnp.where(bnd, 0.0, 1.0)
            D = [x * fz for x in D]
            O = [o * fz for o in O]

            rel = n - n0
            rel = rel - (rel // (NPW // 2)) * (NPW // 2)
            lrow = [xl_b[e, pl.ds(16 * j, 16)] for j in range(8)]
            rrow = [xsl[pl.ds(rel * 128 + 16 * j, 16)] for j in range(8)]
            m = []
            for j in range(8):
                x = lrow[j] + rrow[j]
                m.append(jnp.maximum(x, 0.2 * x))
            for h in range(HEADS):
                u = m[2 * h] * att_regs[2 * h] + m[2 * h + 1] * att_regs[2 * h + 1]
                p = jnp.exp(_hsum_splat(u))
                D[h] = D[h] + p
                O[2 * h] = O[2 * h] + p * lrow[2 * h]
                O[2 * h + 1] = O[2 * h + 1] + p * lrow[2 * h + 1]
            return (n,) + tuple(D) + tuple(O)
        return edge_body

    def issue(c, b):
        cbase = astart + c * K
        pltpu.sync_copy(srcs_h.at[pl.ds(cbase, K)], sidxs[b])
        pltpu.sync_copy(dsts_h.at[pl.ds(cbase, K)], didxs[b].at[pl.ds(0, K)])
        pltpu.async_copy(xl_h.at[sidxs[b]], xlrs[b], sems[b])

    def wait(b):
        pltpu.make_async_copy(xl_h.at[sidxs[b]], xlrs[b], sems[b]).wait()

    def run(out_h, obase):
        emit_row = make_emit(out_h, obase)

        def process(c, b, carry):
            cbase = astart + c * K
            lo = jnp.maximum(s0 - cbase, 0)
            hi = jnp.minimum(K, s1 - cbase)
            return lax.fori_loop(lo, hi, make_edge_body(cbase, b, emit_row),
                                 carry)

        def pair_body(p, carry):
            ca = 2 * p
            cb = ca + 1

            @pl.when(cb < nchunks)
            def _():
                issue(cb, 1)
            wait(0)
            carry = process(ca, 0, carry)

            @pl.when(cb + 1 < nchunks)
            def _():
                issue(cb + 1, 0)

            @pl.when(cb < nchunks)
            def _():
                wait(1)
            return process(cb, 1, carry)

        issue(0, 0)
        init = (n0,) + (zero,) * 12
        npairs = (nchunks + 1) // 2
        carry = lax.fori_loop(0, npairs, pair_body, init)
        emit_row(carry[0], carry[1:5], carry[5:13])

    @pl.when(core == 0)
    def _():
        run(out0_h, 0)

    @pl.when(core == 1)
    def _():
        run(out1_h, NN // 2)


@functools.partial(jax.jit, static_argnames=())
def _noop(x):
    return x


@functools.cache
def _make_edge_call():
    mesh = plsc.VectorSubcoreMesh(core_axis_name="c", subcore_axis_name="s")
    return pl.kernel(
        _edge_kernel_body,
        out_type=[jax.ShapeDtypeStruct((NN // 2 * 128,), jnp.float32),
                  jax.ShapeDtypeStruct((NN // 2 * 128,), jnp.float32)],
        mesh=mesh,
        scratch_types=[
            pltpu.VMEM((56,), jnp.int32),
            pltpu.VMEM((K,), jnp.int32),
            pltpu.VMEM((K,), jnp.int32),
            pltpu.VMEM((K + 16,), jnp.int32),
            pltpu.VMEM((K + 16,), jnp.int32),
            pltpu.VMEM((K, 128), jnp.float32),
            pltpu.VMEM((K, 128), jnp.float32),
            pltpu.VMEM((NPW // 2 * 128,), jnp.float32),
            pltpu.VMEM((128,), jnp.float32),
            pltpu.VMEM((S * 128,), jnp.float32),
            pltpu.SemaphoreType.DMA,
            pltpu.SemaphoreType.DMA,
        ],
    )


RB = 3200


def _enc_body(p_ref, w1, b1, w2, b2, w3, b3, o_ref):
    h0 = jax.nn.silu(jnp.dot(p_ref[...], w1[...],
                             preferred_element_type=jnp.float32) + b1[...])
    h0 = jax.nn.silu(jnp.dot(h0, w2[...],
                             preferred_element_type=jnp.float32) + b2[...])
    o_ref[...] = jnp.dot(h0, w3[...],
                         preferred_element_type=jnp.float32) + b3[...]


def _enc_call(params, w):
    CB = 2560
    grid = (128000 // CB,)
    return pl.pallas_call(
        _enc_body,
        grid=grid,
        in_specs=[
            pl.BlockSpec((32, 4), lambda i: (0, 0)),
            pl.BlockSpec((4, HID), lambda i: (0, 0)),
            pl.BlockSpec((1, HID), lambda i: (0, 0)),
            pl.BlockSpec((HID, HID), lambda i: (0, 0)),
            pl.BlockSpec((1, HID), lambda i: (0, 0)),
            pl.BlockSpec((HID, CB), lambda i: (0, i)),
            pl.BlockSpec((1, CB), lambda i: (0, i)),
        ],
        out_specs=pl.BlockSpec((32, CB), lambda i: (0, i)),
        out_shape=jax.ShapeDtypeStruct((32, 128000), jnp.float32),
    )(params, w['enc_W1'], w['enc_b1'], w['enc_W2'], w['enc_b2'],
      w['enc_W3'], w['enc_b3'])


def _proj_body(h_ref, wl, bl, wr, br, xl_ref, xr_ref):
    h = h_ref[...]
    xl_ref[...] = jnp.dot(h, wl[...], preferred_element_type=jnp.float32) + bl[...]
    xr_ref[...] = jnp.dot(h, wr[...], preferred_element_type=jnp.float32) + br[...]


def _proj_call(h, wl, bl, wr, br):
    grid = (NN // RB,)
    return pl.pallas_call(
        _proj_body,
        grid=grid,
        in_specs=[
            pl.BlockSpec((RB, HID), lambda i: (i, 0)),
            pl.BlockSpec((HID, HID), lambda i: (0, 0)),
            pl.BlockSpec((1, HID), lambda i: (0, 0)),
            pl.BlockSpec((HID, HID), lambda i: (0, 0)),
            pl.BlockSpec((1, HID), lambda i: (0, 0)),
        ],
        out_specs=[
            pl.BlockSpec((RB, HID), lambda i: (i, 0)),
            pl.BlockSpec((RB, HID), lambda i: (i, 0)),
        ],
        out_shape=[
            jax.ShapeDtypeStruct((NN, HID), jnp.float32),
            jax.ShapeDtypeStruct((NN, HID), jnp.float32),
        ],
    )(h, wl, bl, wr, br)


def _post(out_raw, h_res, cb, g, be):
    out = out_raw + cb
    mu = jnp.mean(out, axis=-1, keepdims=True)
    var = jnp.mean((out - mu) ** 2, axis=-1, keepdims=True)
    out = (out - mu) / jnp.sqrt(var + 1e-5) * g + be
    out = jax.nn.silu(out)
    return out + h_res


def _postproj_body(or_ref, hr_ref, cb, g, be, wl, bl, wr, br,
                   h_ref, xl_ref, xr_ref):
    h = _post(or_ref[...], hr_ref[...], cb[...], g[...], be[...])
    h_ref[...] = h
    xl_ref[...] = jnp.dot(h, wl[...], preferred_element_type=jnp.float32) + bl[...]
    xr_ref[...] = jnp.dot(h, wr[...], preferred_element_type=jnp.float32) + br[...]


def _postproj_call(out_raw, h_res, cb, g, be, wl, bl, wr, br):
    grid = (NN // RB,)
    full = lambda r, c: pl.BlockSpec((r, c), lambda i: (0, 0))
    row = pl.BlockSpec((RB, HID), lambda i: (i, 0))
    return pl.pallas_call(
        _postproj_body,
        grid=grid,
        in_specs=[row, row, full(1, HID), full(1, HID), full(1, HID),
                  full(HID, HID), full(1, HID), full(HID, HID), full(1, HID)],
        out_specs=[row, row, row],
        out_shape=[jax.ShapeDtypeStruct((NN, HID), jnp.float32)] * 3,
    )(out_raw, h_res, cb, g, be, wl, bl, wr, br)


def _postdec_body(or_ref, hr_ref, cb, g, be, w1, b1, w2, b2, o_ref):
    h = _post(or_ref[...], hr_ref[...], cb[...], g[...], be[...])
    t = jax.nn.silu(jnp.dot(h, w1[...], preferred_element_type=jnp.float32)
                    + b1[...])
    o_ref[...] = jnp.dot(t, w2[...], preferred_element_type=jnp.float32) + b2[...]


def _postdec_call(out_raw, h_res, cb, g, be, w1, b1, w2, b2):
    grid = (NN // RB,)
    full = lambda r, c: pl.BlockSpec((r, c), lambda i: (0, 0))
    row = pl.BlockSpec((RB, HID), lambda i: (i, 0))
    return pl.pallas_call(
        _postdec_body,
        grid=grid,
        in_specs=[row, row, full(1, HID), full(1, HID), full(1, HID),
                  full(HID, HID), full(1, HID), full(HID, 1), full(1, 1)],
        out_specs=pl.BlockSpec((RB, 1), lambda i: (i, 0)),
        out_shape=jax.ShapeDtypeStruct((NN, 1), jnp.float32),
    )(out_raw, h_res, cb, g, be, w1, b1, w2, b2)


def kernel(params, weights, edge_index):
    w = weights
    loops = jnp.arange(NN, dtype=jnp.int32)
    src = jnp.concatenate([edge_index[0].astype(jnp.int32), loops])
    dst = jnp.concatenate([edge_index[1].astype(jnp.int32), loops])
    key = jnp.sort((dst << 15) | src)
    src_s = key & 0x7FFF
    dst_s = key >> 15
    wb = jnp.searchsorted(dst_s, jnp.arange(0, NN + 1, NPW, dtype=jnp.int32)
                          ).astype(jnp.int32)
    src_sp = jnp.concatenate([src_s, jnp.zeros((K,), jnp.int32)])
    dst_sp = jnp.concatenate([dst_s, jnp.zeros((K,), jnp.int32)])
    wb_pad = jnp.concatenate([wb, jnp.full((23,), ET, jnp.int32)])

    p4 = jnp.pad(params, ((0, 0), (0, 1)))
    w1 = jnp.pad(w['enc_W1'], ((0, 1), (0, 0)))
    hflat = _enc_call(p4, {**w, 'enc_W1': w1,
                           'enc_b1': w['enc_b1'][None, :],
                           'enc_b2': w['enc_b2'][None, :],
                           'enc_b3': w['enc_b3'][None, :]})
    h = hflat.reshape(NN, HID)

    xl, xr = _proj_call(h, w['Wl0'], w['bl0'][None, :],
                        w['Wr0'], w['br0'][None, :])
    for l in range(NLAYERS):
        att_flat = w[f'att{l}'].reshape(128)
        out0, out1 = _make_edge_call()(xl, xr.reshape(NN * 128), src_sp,
                                       dst_sp, wb_pad, att_flat)
        out_raw = jnp.concatenate([out0, out1]).reshape(NN, HID)
        if l < NLAYERS - 1:
            h, xl, xr = _postproj_call(
                out_raw, h, w[f'cb{l}'][None, :], w[f'g{l}'][None, :],
                w[f'be{l}'][None, :], w[f'Wl{l+1}'], w[f'bl{l+1}'][None, :],
                w[f'Wr{l+1}'], w[f'br{l+1}'][None, :])
        else:
            pred = _postdec_call(
                out_raw, h, w[f'cb{l}'][None, :], w[f'g{l}'][None, :],
                w[f'be{l}'][None, :], w['dec_W1'], w['dec_b1'][None, :],
                w['dec_W2'], w['dec_b2'][None, :])
    return pred.reshape(NN)


# speedup vs baseline: 1.4047x; 1.0244x over previous
import functools

import jax
import jax.numpy as jnp
from jax import lax
from jax.experimental import pallas as pl
from jax.experimental.pallas import tpu as pltpu
from jax.experimental.pallas import tpu_sc as plsc

NN = 32000
HID = 128
HEADS = 4
HD = 32
NLAYERS = 4
E = 512000
ET = E + NN
NW = 32
NPW = NN // NW
K = 192
S = 50

_IOTA16 = None


def _iota16():
    return lax.iota(jnp.int32, 16)


def _hsum_splat(v):
    iota = _iota16()
    for k in (8, 4, 2, 1):
        v = v + v.at[iota ^ k].get(mode='promise_in_bounds')
    return v


def _sread(ref, i):
    v = ref[pl.ds(i, 16)]
    return v[0]


def _worker_id():
    return lax.axis_index("c") * 16 + lax.axis_index("s")


def _edge_kernel_body(xl_h, xr_h, srcs_h, dsts_h, wb_h, att_h, out0_h,
                      out1_h, wbv, sidx0, sidx1, didx0, didx1, xlr0, xlr1,
                      xsl, attv, slab, sem0, sem1):
    wid = _worker_id()
    core = wid // 16
    n0 = wid * NPW

    pltpu.sync_copy(wb_h, wbv)
    pltpu.sync_copy(att_h, attv)
    pltpu.sync_copy(xr_h.at[pl.ds(n0 * 128, NPW // 2 * 128)], xsl)

    s0 = _sread(wbv, wid)
    s1 = _sread(wbv, wid + 1)
    astart = (s0 // 8) * 8
    nchunks = (s1 - astart + K - 1) // K
    sems = (sem0, sem1)
    sidxs, didxs = (sidx0, sidx1), (didx0, didx1)
    xlrs = (xlr0, xlr1)

    att_regs = [attv[pl.ds(16 * j, 16)] for j in range(8)]
    zero = jnp.zeros((16,), jnp.float32)
    iota = _iota16()

    def make_emit(out_h, obase):
        def emit_row(n, D, O):
            rel = n - n0
            slot = rel - (rel // S) * S
            for h in range(HEADS):
                inv = 1.0 / (D[h] + 1e-16)
                for jj in range(2):
                    j = 2 * h + jj
                    slab[pl.ds(slot * 128 + 16 * j, 16)] = O[j] * inv

            @pl.when(slot == S - 1)
            def _flush():
                pltpu.sync_copy(
                    slab,
                    out_h.at[pl.ds((n - obase - (S - 1)) * 128, S * 128)])
        return emit_row

    def make_edge_body(cbase, b, emit_row):
        xl_b, d_b = xlrs[b], didxs[b]

        def edge_body(e, carry):
            n = carry[0]
            D = list(carry[1:5])
            O = list(carry[5:13])

            d = d_b[pl.ds(e, 16)][0]
            bnd = d != n

            @pl.when(bnd)
            def _():
                emit_row(n, D, O)

            @pl.when(bnd & (d == n0 + NPW // 2))
            def _():
                pltpu.sync_copy(
                    xr_h.at[pl.ds((n0 + NPW // 2) * 128, NPW // 2 * 128)], xsl)

            n = jnp.where(bnd, d, n)
            fz = jnp.where(bnd, 0.0, 1.0)
            D = [x * fz for x in D]
            O = [o * fz for o in O]

            rel = n - n0
            rel = rel - (rel // (NPW // 2)) * (NPW // 2)
            lrow = [xl_b[e, pl.ds(16 * j, 16)] for j in range(8)]
            rrow = [xsl[pl.ds(rel * 128 + 16 * j, 16)] for j in range(8)]
            m = []
            for j in range(8):
                x = lrow[j] + rrow[j]
                m.append(jnp.maximum(x, 0.2 * x))
            for h in range(HEADS):
                u = m[2 * h] * att_regs[2 * h] + m[2 * h + 1] * att_regs[2 * h + 1]
                p = jnp.exp(_hsum_splat(u))
                D[h] = D[h] + p
                O[2 * h] = O[2 * h] + p * lrow[2 * h]
                O[2 * h + 1] = O[2 * h + 1] + p * lrow[2 * h + 1]
            return (n,) + tuple(D) + tuple(O)
        return edge_body

    def issue(c, b):
        cbase = astart + c * K
        pltpu.sync_copy(srcs_h.at[pl.ds(cbase, K)], sidxs[b])
        pltpu.sync_copy(dsts_h.at[pl.ds(cbase, K)], didxs[b].at[pl.ds(0, K)])
        pltpu.async_copy(xl_h.at[sidxs[b]], xlrs[b], sems[b])

    def wait(b):
        pltpu.make_async_copy(xl_h.at[sidxs[b]], xlrs[b], sems[b]).wait()

    def run(out_h, obase):
        emit_row = make_emit(out_h, obase)

        def process(c, b, carry):
            cbase = astart + c * K
            lo = jnp.maximum(s0 - cbase, 0)
            hi = jnp.minimum(K, s1 - cbase)
            return lax.fori_loop(lo, hi, make_edge_body(cbase, b, emit_row),
                                 carry)

        def pair_body(p, carry):
            ca = 2 * p
            cb = ca + 1

            @pl.when(cb < nchunks)
            def _():
                issue(cb, 1)
            wait(0)
            carry = process(ca, 0, carry)

            @pl.when(cb + 1 < nchunks)
            def _():
                issue(cb + 1, 0)

            @pl.when(cb < nchunks)
            def _():
                wait(1)
            return process(cb, 1, carry)

        issue(0, 0)
        init = (n0,) + (zero,) * 12
        npairs = (nchunks + 1) // 2
        carry = lax.fori_loop(0, npairs, pair_body, init)
        emit_row(carry[0], carry[1:5], carry[5:13])

    @pl.when(core == 0)
    def _():
        run(out0_h, 0)

    @pl.when(core == 1)
    def _():
        run(out1_h, NN // 2)


@functools.partial(jax.jit, static_argnames=())
def _noop(x):
    return x


@functools.cache
def _make_edge_call():
    mesh = plsc.VectorSubcoreMesh(core_axis_name="c", subcore_axis_name="s")
    return pl.kernel(
        _edge_kernel_body,
        out_type=[jax.ShapeDtypeStruct((NN // 2 * 128,), jnp.float32),
                  jax.ShapeDtypeStruct((NN // 2 * 128,), jnp.float32)],
        mesh=mesh,
        scratch_types=[
            pltpu.VMEM((56,), jnp.int32),
            pltpu.VMEM((K,), jnp.int32),
            pltpu.VMEM((K,), jnp.int32),
            pltpu.VMEM((K + 16,), jnp.int32),
            pltpu.VMEM((K + 16,), jnp.int32),
            pltpu.VMEM((K, 128), jnp.float32),
            pltpu.VMEM((K, 128), jnp.float32),
            pltpu.VMEM((NPW // 2 * 128,), jnp.float32),
            pltpu.VMEM((128,), jnp.float32),
            pltpu.VMEM((S * 128,), jnp.float32),
            pltpu.SemaphoreType.DMA,
            pltpu.SemaphoreType.DMA,
        ],
    )


RB = 3200


def _enc_body(p_ref, w1, b1, w2, b2, w3, b3, o_ref):
    h0 = jax.nn.silu(jnp.dot(p_ref[...], w1[...],
                             preferred_element_type=jnp.float32) + b1[...])
    h0 = jax.nn.silu(jnp.dot(h0, w2[...],
                             preferred_element_type=jnp.float32) + b2[...])
    o_ref[...] = jnp.dot(h0, w3[...],
                         preferred_element_type=jnp.float32) + b3[...]


def _enc_call(params, w):
    CB = 2560
    grid = (128000 // CB,)
    return pl.pallas_call(
        _enc_body,
        grid=grid,
        in_specs=[
            pl.BlockSpec((32, 4), lambda i: (0, 0)),
            pl.BlockSpec((4, HID), lambda i: (0, 0)),
            pl.BlockSpec((1, HID), lambda i: (0, 0)),
            pl.BlockSpec((HID, HID), lambda i: (0, 0)),
            pl.BlockSpec((1, HID), lambda i: (0, 0)),
            pl.BlockSpec((HID, CB), lambda i: (0, i)),
            pl.BlockSpec((1, CB), lambda i: (0, i)),
        ],
        out_specs=pl.BlockSpec((32, CB), lambda i: (0, i)),
        out_shape=jax.ShapeDtypeStruct((32, 128000), jnp.float32),
    )(params, w['enc_W1'], w['enc_b1'], w['enc_W2'], w['enc_b2'],
      w['enc_W3'], w['enc_b3'])


def _proj_body(h_ref, wl, bl, wr, br, xl_ref, xr_ref):
    h = h_ref[...]
    xl_ref[...] = jnp.dot(h, wl[...], preferred_element_type=jnp.float32) + bl[...]
    xr_ref[...] = jnp.dot(h, wr[...], preferred_element_type=jnp.float32) + br[...]


def _proj_call(h, wl, bl, wr, br):
    grid = (NN // RB,)
    return pl.pallas_call(
        _proj_body,
        grid=grid,
        in_specs=[
            pl.BlockSpec((RB, HID), lambda i: (i, 0)),
            pl.BlockSpec((HID, HID), lambda i: (0, 0)),
            pl.BlockSpec((1, HID), lambda i: (0, 0)),
            pl.BlockSpec((HID, HID), lambda i: (0, 0)),
            pl.BlockSpec((1, HID), lambda i: (0, 0)),
        ],
        out_specs=[
            pl.BlockSpec((RB, HID), lambda i: (i, 0)),
            pl.BlockSpec((RB, HID), lambda i: (i, 0)),
        ],
        out_shape=[
            jax.ShapeDtypeStruct((NN, HID), jnp.float32),
            jax.ShapeDtypeStruct((NN, HID), jnp.float32),
        ],
    )(h, wl, bl, wr, br)


def _post(out_raw, h_res, cb, g, be):
    out = out_raw + cb
    mu = jnp.mean(out, axis=-1, keepdims=True)
    var = jnp.mean((out - mu) ** 2, axis=-1, keepdims=True)
    out = (out - mu) / jnp.sqrt(var + 1e-5) * g + be
    out = jax.nn.silu(out)
    return out + h_res


def _postproj_body(or_ref, hr_ref, cb, g, be, wl, bl, wr, br,
                   h_ref, xl_ref, xr_ref):
    h = _post(or_ref[...], hr_ref[...], cb[...], g[...], be[...])
    h_ref[...] = h
    xl_ref[...] = jnp.dot(h, wl[...], preferred_element_type=jnp.float32) + bl[...]
    xr_ref[...] = jnp.dot(h, wr[...], preferred_element_type=jnp.float32) + br[...]


def _postproj_call(out_raw, h_res, cb, g, be, wl, bl, wr, br):
    grid = (NN // RB,)
    full = lambda r, c: pl.BlockSpec((r, c), lambda i: (0, 0))
    row = pl.BlockSpec((RB, HID), lambda i: (i, 0))
    return pl.pallas_call(
        _postproj_body,
        grid=grid,
        in_specs=[row, row, full(1, HID), full(1, HID), full(1, HID),
                  full(HID, HID), full(1, HID), full(HID, HID), full(1, HID)],
        out_specs=[row, row, row],
        out_shape=[jax.ShapeDtypeStruct((NN, HID), jnp.float32)] * 3,
    )(out_raw, h_res, cb, g, be, wl, bl, wr, br)


def _postdec_body(or_ref, hr_ref, cb, g, be, w1, b1, w2, b2, o_ref):
    h = _post(or_ref[...], hr_ref[...], cb[...], g[...], be[...])
    t = jax.nn.silu(jnp.dot(h, w1[...], preferred_element_type=jnp.float32)
                    + b1[...])
    o_ref[...] = jnp.dot(t, w2[...], preferred_element_type=jnp.float32) + b2[...]


def _postdec_call(out_raw, h_res, cb, g, be, w1, b1, w2, b2):
    grid = (NN // RB,)
    full = lambda r, c: pl.BlockSpec((r, c), lambda i: (0, 0))
    row = pl.BlockSpec((RB, HID), lambda i: (i, 0))
    return pl.pallas_call(
        _postdec_body,
        grid=grid,
        in_specs=[row, row, full(1, HID), full(1, HID), full(1, HID),
                  full(HID, HID), full(1, HID), full(HID, 1), full(1, 1)],
        out_specs=pl.BlockSpec((RB, 1), lambda i: (i, 0)),
        out_shape=jax.ShapeDtypeStruct((NN, 1), jnp.float32),
    )(out_raw, h_res, cb, g, be, w1, b1, w2, b2)


def kernel(params, weights, edge_index):
    w = weights
    loops = jnp.arange(NN, dtype=jnp.int32)
    src = jnp.concatenate([edge_index[0].astype(jnp.int32), loops])
    dst = jnp.concatenate([edge_index[1].astype(jnp.int32), loops])
    key = jnp.sort((dst << 15) | src)
    src_s = key & 0x7FFF
    dst_s = key >> 15
    wb = jnp.searchsorted(dst_s, jnp.arange(0, NN + 1, NPW, dtype=jnp.int32)
                          ).astype(jnp.int32)
    src_sp = jnp.concatenate([src_s, jnp.zeros((K,), jnp.int32)])
    dst_sp = jnp.concatenate([dst_s, jnp.zeros((K,), jnp.int32)])
    wb_pad = jnp.concatenate([wb, jnp.full((23,), ET, jnp.int32)])

    p4 = jnp.pad(params, ((0, 0), (0, 1)))
    w1 = jnp.pad(w['enc_W1'], ((0, 1), (0, 0)))
    hflat = _enc_call(p4, {**w, 'enc_W1': w1,
                           'enc_b1': w['enc_b1'][None, :],
                           'enc_b2': w['enc_b2'][None, :],
                           'enc_b3': w['enc_b3'][None, :]})
    h = hflat.reshape(NN, HID)

    xl, xr = _proj_call(h, w['Wl0'], w['bl0'][None, :],
                        w['Wr0'], w['br0'][None, :])
    for l in range(NLAYERS):
        att_flat = w[f'att{l}'].reshape(128)
        out0, out1 = _make_edge_call()(xl, xr.reshape(NN * 128), src_sp,
                                       dst_sp, wb_pad, att_flat)
        out_raw = jnp.concatenate([out0, out1]).reshape(NN, HID)
        if l < NLAYERS - 1:
            h, xl, xr = _postproj_call(
                out_raw, h, w[f'cb{l}'][None, :], w[f'g{l}'][None, :],
                w[f'be{l}'][None, :], w[f'Wl{l+1}'], w[f'bl{l+1}'][None, :],
                w[f'Wr{l+1}'], w[f'br{l+1}'][None, :])
        else:
            pred = _postdec_call(
                out_raw, h, w[f'cb{l}'][None, :], w[f'g{l}'][None, :],
                w[f'be{l}'][None, :], w['dec_W1'], w['dec_b1'][None, :],
                w['dec_W2'], w['dec_b2'][None, :])
    return pred.reshape(NN)


# speedup vs baseline: 1.5101x; 1.0750x over previous
import functools

import jax
import jax.numpy as jnp
from jax import lax
from jax.experimental import pallas as pl
from jax.experimental.pallas import tpu as pltpu
from jax.experimental.pallas import tpu_sc as plsc

NN = 32000
HID = 128
HEADS = 4
HD = 32
NLAYERS = 4
E = 512000
ET = E + NN
NW = 32
NPW = NN // NW
K = 192
S = 50

_IOTA16 = None


def _iota16():
    return lax.iota(jnp.int32, 16)


def _hsum_splat(v):
    iota = _iota16()
    for k in (8, 4, 2, 1):
        v = v + v.at[iota ^ k].get(mode='promise_in_bounds')
    return v


def _sread(ref, i):
    v = ref[pl.ds(i, 16)]
    return v[0]


def _worker_id():
    return lax.axis_index("c") * 16 + lax.axis_index("s")


def _edge_kernel_body(xl_h, xr_h, srcs_h, dsts_h, wb_h, att_h, out0_h,
                      out1_h, wbv, sidx0, sidx1, didx0, didx1, xlr0, xlr1,
                      xsl, attv, slab, sem0, sem1):
    wid = _worker_id()
    core = wid // 16
    n0 = wid * NPW

    pltpu.sync_copy(wb_h, wbv)
    pltpu.sync_copy(att_h, attv)
    pltpu.sync_copy(xr_h.at[pl.ds(n0 * 128, NPW // 2 * 128)], xsl)

    s0 = _sread(wbv, wid)
    s1 = _sread(wbv, wid + 1)
    astart = (s0 // 8) * 8
    nchunks = (s1 - astart + K - 1) // K
    sems = (sem0, sem1)
    sidxs, didxs = (sidx0, sidx1), (didx0, didx1)
    xlrs = (xlr0, xlr1)

    att_regs = [attv[pl.ds(16 * j, 16)] for j in range(8)]
    zero = jnp.zeros((16,), jnp.float32)
    iota = _iota16()

    def make_emit(out_h, obase):
        def emit_row(n, D, O):
            rel = n - n0
            slot = rel - (rel // S) * S
            for h in range(HEADS):
                inv = 1.0 / (D[h] + 1e-16)
                for jj in range(2):
                    j = 2 * h + jj
                    slab[pl.ds(slot * 128 + 16 * j, 16)] = O[j] * inv

            @pl.when(slot == S - 1)
            def _flush():
                pltpu.sync_copy(
                    slab,
                    out_h.at[pl.ds((n - obase - (S - 1)) * 128, S * 128)])
        return emit_row

    def make_edge_body(cbase, b, emit_row):
        xl_b, d_b = xlrs[b], didxs[b]

        def edge_body(e, carry):
            dprev = carry[0]
            D = list(carry[1:5])
            O = list(carry[5:13])

            w = d_b[pl.ds(e, 16)][0]
            d = w & 0x7FFF
            off = lax.shift_right_logical(w, 15)
            bnd = d != dprev

            @pl.when(bnd)
            def _():
                emit_row(dprev, D, O)

            @pl.when(bnd & (d == n0 + NPW // 2))
            def _():
                pltpu.sync_copy(
                    xr_h.at[pl.ds((n0 + NPW // 2) * 128, NPW // 2 * 128)], xsl)

            fz = jnp.where(bnd, 0.0, 1.0)
            D = [x * fz for x in D]
            O = [o * fz for o in O]

            lrow = [xl_b[e, pl.ds(16 * j, 16)] for j in range(8)]
            rrow = [xsl[pl.ds(off + 16 * j, 16)] for j in range(8)]
            m = []
            for j in range(8):
                x = lrow[j] + rrow[j]
                m.append(jnp.maximum(x, 0.2 * x))
            for h in range(HEADS):
                u = m[2 * h] * att_regs[2 * h] + m[2 * h + 1] * att_regs[2 * h + 1]
                p = jnp.exp(_hsum_splat(u))
                D[h] = D[h] + p
                O[2 * h] = O[2 * h] + p * lrow[2 * h]
                O[2 * h + 1] = O[2 * h + 1] + p * lrow[2 * h + 1]
            return (d,) + tuple(D) + tuple(O)
        return edge_body

    def issue(c, b):
        cbase = astart + c * K
        pltpu.sync_copy(srcs_h.at[pl.ds(cbase, K)], sidxs[b])
        pltpu.sync_copy(dsts_h.at[pl.ds(cbase, K)], didxs[b].at[pl.ds(0, K)])
        pltpu.async_copy(xl_h.at[sidxs[b]], xlrs[b], sems[b])

    def wait(b):
        pltpu.make_async_copy(xl_h.at[sidxs[b]], xlrs[b], sems[b]).wait()

    def run(out_h, obase):
        emit_row = make_emit(out_h, obase)

        def process(c, b, carry):
            cbase = astart + c * K
            lo = jnp.maximum(s0 - cbase, 0)
            hi = jnp.minimum(K, s1 - cbase)
            return lax.fori_loop(lo, hi, make_edge_body(cbase, b, emit_row),
                                 carry)

        def pair_body(p, carry):
            ca = 2 * p
            cb = ca + 1

            @pl.when(cb < nchunks)
            def _():
                issue(cb, 1)
            wait(0)
            carry = process(ca, 0, carry)

            @pl.when(cb + 1 < nchunks)
            def _():
                issue(cb + 1, 0)

            @pl.when(cb < nchunks)
            def _():
                wait(1)
            return process(cb, 1, carry)

        issue(0, 0)
        init = (n0,) + (zero,) * 12
        npairs = (nchunks + 1) // 2
        carry = lax.fori_loop(0, npairs, pair_body, init)
        emit_row(carry[0], carry[1:5], carry[5:13])

    @pl.when(core == 0)
    def _():
        run(out0_h, 0)

    @pl.when(core == 1)
    def _():
        run(out1_h, NN // 2)


@functools.partial(jax.jit, static_argnames=())
def _noop(x):
    return x


@functools.cache
def _make_edge_call():
    mesh = plsc.VectorSubcoreMesh(core_axis_name="c", subcore_axis_name="s")
    return pl.kernel(
        _edge_kernel_body,
        out_type=[jax.ShapeDtypeStruct((NN // 2 * 128,), jnp.float32),
                  jax.ShapeDtypeStruct((NN // 2 * 128,), jnp.float32)],
        mesh=mesh,
        scratch_types=[
            pltpu.VMEM((56,), jnp.int32),
            pltpu.VMEM((K,), jnp.int32),
            pltpu.VMEM((K,), jnp.int32),
            pltpu.VMEM((K + 16,), jnp.int32),
            pltpu.VMEM((K + 16,), jnp.int32),
            pltpu.VMEM((K, 128), jnp.float32),
            pltpu.VMEM((K, 128), jnp.float32),
            pltpu.VMEM((NPW // 2 * 128,), jnp.float32),
            pltpu.VMEM((128,), jnp.float32),
            pltpu.VMEM((S * 128,), jnp.float32),
            pltpu.SemaphoreType.DMA,
            pltpu.SemaphoreType.DMA,
        ],
    )


RB = 3200


def _enc_body(p_ref, w1, b1, w2, b2, w3, b3, o_ref):
    h0 = jax.nn.silu(jnp.dot(p_ref[...], w1[...],
                             preferred_element_type=jnp.float32) + b1[...])
    h0 = jax.nn.silu(jnp.dot(h0, w2[...],
                             preferred_element_type=jnp.float32) + b2[...])
    o_ref[...] = jnp.dot(h0, w3[...],
                         preferred_element_type=jnp.float32) + b3[...]


def _enc_call(params, w):
    CB = 2560
    grid = (128000 // CB,)
    return pl.pallas_call(
        _enc_body,
        grid=grid,
        in_specs=[
            pl.BlockSpec((32, 4), lambda i: (0, 0)),
            pl.BlockSpec((4, HID), lambda i: (0, 0)),
            pl.BlockSpec((1, HID), lambda i: (0, 0)),
            pl.BlockSpec((HID, HID), lambda i: (0, 0)),
            pl.BlockSpec((1, HID), lambda i: (0, 0)),
            pl.BlockSpec((HID, CB), lambda i: (0, i)),
            pl.BlockSpec((1, CB), lambda i: (0, i)),
        ],
        out_specs=pl.BlockSpec((32, CB), lambda i: (0, i)),
        out_shape=jax.ShapeDtypeStruct((32, 128000), jnp.float32),
    )(params, w['enc_W1'], w['enc_b1'], w['enc_W2'], w['enc_b2'],
      w['enc_W3'], w['enc_b3'])


def _proj_body(h_ref, wl, bl, wr, br, xl_ref, xr_ref):
    h = h_ref[...]
    xl_ref[...] = jnp.dot(h, wl[...], preferred_element_type=jnp.float32) + bl[...]
    xr_ref[...] = jnp.dot(h, wr[...], preferred_element_type=jnp.float32) + br[...]


def _proj_call(h, wl, bl, wr, br):
    grid = (NN // RB,)
    return pl.pallas_call(
        _proj_body,
        grid=grid,
        in_specs=[
            pl.BlockSpec((RB, HID), lambda i: (i, 0)),
            pl.BlockSpec((HID, HID), lambda i: (0, 0)),
            pl.BlockSpec((1, HID), lambda i: (0, 0)),
            pl.BlockSpec((HID, HID), lambda i: (0, 0)),
            pl.BlockSpec((1, HID), lambda i: (0, 0)),
        ],
        out_specs=[
            pl.BlockSpec((RB, HID), lambda i: (i, 0)),
            pl.BlockSpec((RB, HID), lambda i: (i, 0)),
        ],
        out_shape=[
            jax.ShapeDtypeStruct((NN, HID), jnp.float32),
            jax.ShapeDtypeStruct((NN, HID), jnp.float32),
        ],
    )(h, wl, bl, wr, br)


def _post(out_raw, h_res, cb, g, be):
    out = out_raw + cb
    mu = jnp.mean(out, axis=-1, keepdims=True)
    var = jnp.mean((out - mu) ** 2, axis=-1, keepdims=True)
    out = (out - mu) / jnp.sqrt(var + 1e-5) * g + be
    out = jax.nn.silu(out)
    return out + h_res


def _postproj_body(or_ref, hr_ref, cb, g, be, wl, bl, wr, br,
                   h_ref, xl_ref, xr_ref):
    h = _post(or_ref[...], hr_ref[...], cb[...], g[...], be[...])
    h_ref[...] = h
    xl_ref[...] = jnp.dot(h, wl[...], preferred_element_type=jnp.float32) + bl[...]
    xr_ref[...] = jnp.dot(h, wr[...], preferred_element_type=jnp.float32) + br[...]


def _postproj_call(out_raw, h_res, cb, g, be, wl, bl, wr, br):
    grid = (NN // RB,)
    full = lambda r, c: pl.BlockSpec((r, c), lambda i: (0, 0))
    row = pl.BlockSpec((RB, HID), lambda i: (i, 0))
    return pl.pallas_call(
        _postproj_body,
        grid=grid,
        in_specs=[row, row, full(1, HID), full(1, HID), full(1, HID),
                  full(HID, HID), full(1, HID), full(HID, HID), full(1, HID)],
        out_specs=[row, row, row],
        out_shape=[jax.ShapeDtypeStruct((NN, HID), jnp.float32)] * 3,
    )(out_raw, h_res, cb, g, be, wl, bl, wr, br)


def _postdec_body(or_ref, hr_ref, cb, g, be, w1, b1, w2, b2, o_ref):
    h = _post(or_ref[...], hr_ref[...], cb[...], g[...], be[...])
    t = jax.nn.silu(jnp.dot(h, w1[...], preferred_element_type=jnp.float32)
                    + b1[...])
    o_ref[...] = jnp.dot(t, w2[...], preferred_element_type=jnp.float32) + b2[...]


def _postdec_call(out_raw, h_res, cb, g, be, w1, b1, w2, b2):
    grid = (NN // RB,)
    full = lambda r, c: pl.BlockSpec((r, c), lambda i: (0, 0))
    row = pl.BlockSpec((RB, HID), lambda i: (i, 0))
    return pl.pallas_call(
        _postdec_body,
        grid=grid,
        in_specs=[row, row, full(1, HID), full(1, HID), full(1, HID),
                  full(HID, HID), full(1, HID), full(HID, 1), full(1, 1)],
        out_specs=pl.BlockSpec((RB, 1), lambda i: (i, 0)),
        out_shape=jax.ShapeDtypeStruct((NN, 1), jnp.float32),
    )(out_raw, h_res, cb, g, be, w1, b1, w2, b2)


def kernel(params, weights, edge_index):
    w = weights
    loops = jnp.arange(NN, dtype=jnp.int32)
    src = jnp.concatenate([edge_index[0].astype(jnp.int32), loops])
    dst = jnp.concatenate([edge_index[1].astype(jnp.int32), loops])
    key = jnp.sort((dst << 15) | src)
    src_s = key & 0x7FFF
    dst_s = key >> 15
    wb = jnp.searchsorted(dst_s, jnp.arange(0, NN + 1, NPW, dtype=jnp.int32)
                          ).astype(jnp.int32)
    src_sp = jnp.concatenate([src_s, jnp.zeros((K,), jnp.int32)])
    dpack = dst_s | ((dst_s % (NPW // 2)) * 128 << 15)
    dst_sp = jnp.concatenate([dpack, jnp.zeros((K,), jnp.int32)])
    wb_pad = jnp.concatenate([wb, jnp.full((23,), ET, jnp.int32)])

    p4 = jnp.pad(params, ((0, 0), (0, 1)))
    w1 = jnp.pad(w['enc_W1'], ((0, 1), (0, 0)))
    hflat = _enc_call(p4, {**w, 'enc_W1': w1,
                           'enc_b1': w['enc_b1'][None, :],
                           'enc_b2': w['enc_b2'][None, :],
                           'enc_b3': w['enc_b3'][None, :]})
    h = hflat.reshape(NN, HID)

    xl, xr = _proj_call(h, w['Wl0'], w['bl0'][None, :],
                        w['Wr0'], w['br0'][None, :])
    for l in range(NLAYERS):
        att_flat = w[f'att{l}'].reshape(128)
        out0, out1 = _make_edge_call()(xl, xr.reshape(NN * 128), src_sp,
                                       dst_sp, wb_pad, att_flat)
        out_raw = jnp.concatenate([out0, out1]).reshape(NN, HID)
        if l < NLAYERS - 1:
            h, xl, xr = _postproj_call(
                out_raw, h, w[f'cb{l}'][None, :], w[f'g{l}'][None, :],
                w[f'be{l}'][None, :], w[f'Wl{l+1}'], w[f'bl{l+1}'][None, :],
                w[f'Wr{l+1}'], w[f'br{l+1}'][None, :])
        else:
            pred = _postdec_call(
                out_raw, h, w[f'cb{l}'][None, :], w[f'g{l}'][None, :],
                w[f'be{l}'][None, :], w['dec_W1'], w['dec_b1'][None, :],
                w['dec_W2'], w['dec_b2'][None, :])
    return pred.reshape(NN)


# speedup vs baseline: 1.5849x; 1.0495x over previous
import functools

import jax
import jax.numpy as jnp
from jax import lax
from jax.experimental import pallas as pl
from jax.experimental.pallas import tpu as pltpu
from jax.experimental.pallas import tpu_sc as plsc

NN = 32000
HID = 128
HEADS = 4
HD = 32
NLAYERS = 4
E = 512000
ET = E + NN
NW = 32
NPW = NN // NW
K = 192
S = 50

_IOTA16 = None


def _iota16():
    return lax.iota(jnp.int32, 16)


def _hsum_splat(v):
    iota = _iota16()
    for k in (8, 4, 2, 1):
        v = v + v.at[iota ^ k].get(mode='promise_in_bounds')
    return v


def _sread(ref, i):
    v = ref[pl.ds(i, 16)]
    return v[0]


def _worker_id():
    return lax.axis_index("c") * 16 + lax.axis_index("s")


def _edge_kernel_body(xl_h, xr_h, srcs_h, dsts_h, wb_h, att_h, out0_h,
                      out1_h, wbv, sidx0, sidx1, didx0, didx1, xlr0, xlr1,
                      xsl, attv, slab, sem0, sem1):
    wid = _worker_id()
    core = wid // 16
    n0 = wid * NPW

    pltpu.sync_copy(wb_h, wbv)
    pltpu.sync_copy(att_h, attv)
    pltpu.sync_copy(xr_h.at[pl.ds(n0 * 128, NPW // 2 * 128)], xsl)

    s0 = _sread(wbv, wid)
    s1 = _sread(wbv, wid + 1)
    astart = (s0 // 8) * 8
    nchunks = (s1 - astart + K - 1) // K
    sems = (sem0, sem1)
    sidxs, didxs = (sidx0, sidx1), (didx0, didx1)
    xlrs = (xlr0, xlr1)

    att_regs = [attv[pl.ds(16 * j, 16)] for j in range(8)]
    zero = jnp.zeros((16,), jnp.float32)
    iota = _iota16()

    def make_emit(out_h, obase):
        def emit_row(n, D, O):
            rel = n - n0
            slot = rel - (rel // S) * S
            for h in range(HEADS):
                inv = 1.0 / (D[h] + 1e-16)
                for jj in range(2):
                    j = 2 * h + jj
                    slab[pl.ds(slot * 128 + 16 * j, 16)] = O[j] * inv

            @pl.when(slot == S - 1)
            def _flush():
                pltpu.sync_copy(
                    slab,
                    out_h.at[pl.ds((n - obase - (S - 1)) * 128, S * 128)])
        return emit_row

    def make_edge_body(cbase, b, emit_row):
        xl_b, d_b = xlrs[b], didxs[b]

        def edge_step(e, carry):
            dprev = carry[0]
            D = list(carry[1:5])
            O = list(carry[5:13])

            e_g = cbase + e
            valid = (e_g >= s0) & (e_g < s1)
            w = d_b[pl.ds(e, 16)][0]
            d = w & 0x7FFF
            off = lax.shift_right_logical(w, 15)
            bnd = (d != dprev) & valid

            @pl.when(bnd)
            def _():
                emit_row(dprev, D, O)

            @pl.when(bnd & (d == n0 + NPW // 2))
            def _():
                pltpu.sync_copy(
                    xr_h.at[pl.ds((n0 + NPW // 2) * 128, NPW // 2 * 128)], xsl)

            fz = jnp.where(bnd, 0.0, 1.0)
            vf = jnp.where(valid, 1.0, 0.0)
            D = [x * fz for x in D]
            O = [o * fz for o in O]

            lrow = [xl_b[e, pl.ds(16 * j, 16)] for j in range(8)]
            rrow = [xsl[pl.ds(off + 16 * j, 16)] for j in range(8)]
            m = []
            for j in range(8):
                x = lrow[j] + rrow[j]
                m.append(jnp.maximum(x, 0.2 * x))
            for h in range(HEADS):
                u = m[2 * h] * att_regs[2 * h] + m[2 * h + 1] * att_regs[2 * h + 1]
                p = jnp.exp(_hsum_splat(u) * vf) * vf
                D[h] = D[h] + p
                O[2 * h] = O[2 * h] + p * lrow[2 * h]
                O[2 * h + 1] = O[2 * h + 1] + p * lrow[2 * h + 1]
            dnew = jnp.where(valid, d, dprev)
            return (dnew,) + tuple(D) + tuple(O)

        def edge_body(p2, carry):
            return edge_step(2 * p2 + 1, edge_step(2 * p2, carry))
        return edge_body

    def issue(c, b):
        cbase = astart + c * K
        pltpu.sync_copy(srcs_h.at[pl.ds(cbase, K)], sidxs[b])
        pltpu.sync_copy(dsts_h.at[pl.ds(cbase, K)], didxs[b].at[pl.ds(0, K)])
        pltpu.async_copy(xl_h.at[sidxs[b]], xlrs[b], sems[b])

    def wait(b):
        pltpu.make_async_copy(xl_h.at[sidxs[b]], xlrs[b], sems[b]).wait()

    def run(out_h, obase):
        emit_row = make_emit(out_h, obase)

        def process(c, b, carry):
            cbase = astart + c * K
            return lax.fori_loop(0, K // 2, make_edge_body(cbase, b, emit_row),
                                 carry)

        def pair_body(p, carry):
            ca = 2 * p
            cb = ca + 1

            @pl.when(cb < nchunks)
            def _():
                issue(cb, 1)
            wait(0)
            carry = process(ca, 0, carry)

            @pl.when(cb + 1 < nchunks)
            def _():
                issue(cb + 1, 0)

            @pl.when(cb < nchunks)
            def _():
                wait(1)
            return process(cb, 1, carry)

        issue(0, 0)
        init = (n0,) + (zero,) * 12
        npairs = (nchunks + 1) // 2
        carry = lax.fori_loop(0, npairs, pair_body, init)
        emit_row(carry[0], carry[1:5], carry[5:13])

    @pl.when(core == 0)
    def _():
        run(out0_h, 0)

    @pl.when(core == 1)
    def _():
        run(out1_h, NN // 2)


@functools.partial(jax.jit, static_argnames=())
def _noop(x):
    return x


@functools.cache
def _make_edge_call():
    mesh = plsc.VectorSubcoreMesh(core_axis_name="c", subcore_axis_name="s")
    return pl.kernel(
        _edge_kernel_body,
        out_type=[jax.ShapeDtypeStruct((NN // 2 * 128,), jnp.float32),
                  jax.ShapeDtypeStruct((NN // 2 * 128,), jnp.float32)],
        mesh=mesh,
        scratch_types=[
            pltpu.VMEM((56,), jnp.int32),
            pltpu.VMEM((K,), jnp.int32),
            pltpu.VMEM((K,), jnp.int32),
            pltpu.VMEM((K + 16,), jnp.int32),
            pltpu.VMEM((K + 16,), jnp.int32),
            pltpu.VMEM((K, 128), jnp.float32),
            pltpu.VMEM((K, 128), jnp.float32),
            pltpu.VMEM((NPW // 2 * 128,), jnp.float32),
            pltpu.VMEM((128,), jnp.float32),
            pltpu.VMEM((S * 128,), jnp.float32),
            pltpu.SemaphoreType.DMA,
            pltpu.SemaphoreType.DMA,
        ],
    )


RB = 3200


def _enc_body(p_ref, w1, b1, w2, b2, w3, b3, o_ref):
    h0 = jax.nn.silu(jnp.dot(p_ref[...], w1[...],
                             preferred_element_type=jnp.float32) + b1[...])
    h0 = jax.nn.silu(jnp.dot(h0, w2[...],
                             preferred_element_type=jnp.float32) + b2[...])
    o_ref[...] = jnp.dot(h0, w3[...],
                         preferred_element_type=jnp.float32) + b3[...]


def _enc_call(params, w):
    CB = 2560
    grid = (128000 // CB,)
    return pl.pallas_call(
        _enc_body,
        grid=grid,
        in_specs=[
            pl.BlockSpec((32, 4), lambda i: (0, 0)),
            pl.BlockSpec((4, HID), lambda i: (0, 0)),
            pl.BlockSpec((1, HID), lambda i: (0, 0)),
            pl.BlockSpec((HID, HID), lambda i: (0, 0)),
            pl.BlockSpec((1, HID), lambda i: (0, 0)),
            pl.BlockSpec((HID, CB), lambda i: (0, i)),
            pl.BlockSpec((1, CB), lambda i: (0, i)),
        ],
        out_specs=pl.BlockSpec((32, CB), lambda i: (0, i)),
        out_shape=jax.ShapeDtypeStruct((32, 128000), jnp.float32),
    )(params, w['enc_W1'], w['enc_b1'], w['enc_W2'], w['enc_b2'],
      w['enc_W3'], w['enc_b3'])


def _proj_body(h_ref, wl, bl, wr, br, xl_ref, xr_ref):
    h = h_ref[...]
    xl_ref[...] = jnp.dot(h, wl[...], preferred_element_type=jnp.float32) + bl[...]
    xr_ref[...] = jnp.dot(h, wr[...], preferred_element_type=jnp.float32) + br[...]


def _proj_call(h, wl, bl, wr, br):
    grid = (NN // RB,)
    return pl.pallas_call(
        _proj_body,
        grid=grid,
        in_specs=[
            pl.BlockSpec((RB, HID), lambda i: (i, 0)),
            pl.BlockSpec((HID, HID), lambda i: (0, 0)),
            pl.BlockSpec((1, HID), lambda i: (0, 0)),
            pl.BlockSpec((HID, HID), lambda i: (0, 0)),
            pl.BlockSpec((1, HID), lambda i: (0, 0)),
        ],
        out_specs=[
            pl.BlockSpec((RB, HID), lambda i: (i, 0)),
            pl.BlockSpec((RB, HID), lambda i: (i, 0)),
        ],
        out_shape=[
            jax.ShapeDtypeStruct((NN, HID), jnp.float32),
            jax.ShapeDtypeStruct((NN, HID), jnp.float32),
        ],
    )(h, wl, bl, wr, br)


def _post(out_raw, h_res, cb, g, be):
    out = out_raw + cb
    mu = jnp.mean(out, axis=-1, keepdims=True)
    var = jnp.mean((out - mu) ** 2, axis=-1, keepdims=True)
    out = (out - mu) / jnp.sqrt(var + 1e-5) * g + be
    out = jax.nn.silu(out)
    return out + h_res


def _postproj_body(or_ref, hr_ref, cb, g, be, wl, bl, wr, br,
                   h_ref, xl_ref, xr_ref):
    h = _post(or_ref[...], hr_ref[...], cb[...], g[...], be[...])
    h_ref[...] = h
    xl_ref[...] = jnp.dot(h, wl[...], preferred_element_type=jnp.float32) + bl[...]
    xr_ref[...] = jnp.dot(h, wr[...], preferred_element_type=jnp.float32) + br[...]


def _postproj_call(out_raw, h_res, cb, g, be, wl, bl, wr, br):
    grid = (NN // RB,)
    full = lambda r, c: pl.BlockSpec((r, c), lambda i: (0, 0))
    row = pl.BlockSpec((RB, HID), lambda i: (i, 0))
    return pl.pallas_call(
        _postproj_body,
        grid=grid,
        in_specs=[row, row, full(1, HID), full(1, HID), full(1, HID),
                  full(HID, HID), full(1, HID), full(HID, HID), full(1, HID)],
        out_specs=[row, row, row],
        out_shape=[jax.ShapeDtypeStruct((NN, HID), jnp.float32)] * 3,
    )(out_raw, h_res, cb, g, be, wl, bl, wr, br)


def _postdec_body(or_ref, hr_ref, cb, g, be, w1, b1, w2, b2, o_ref):
    h = _post(or_ref[...], hr_ref[...], cb[...], g[...], be[...])
    t = jax.nn.silu(jnp.dot(h, w1[...], preferred_element_type=jnp.float32)
                    + b1[...])
    o_ref[...] = jnp.dot(t, w2[...], preferred_element_type=jnp.float32) + b2[...]


def _postdec_call(out_raw, h_res, cb, g, be, w1, b1, w2, b2):
    grid = (NN // RB,)
    full = lambda r, c: pl.BlockSpec((r, c), lambda i: (0, 0))
    row = pl.BlockSpec((RB, HID), lambda i: (i, 0))
    return pl.pallas_call(
        _postdec_body,
        grid=grid,
        in_specs=[row, row, full(1, HID), full(1, HID), full(1, HID),
                  full(HID, HID), full(1, HID), full(HID, 1), full(1, 1)],
        out_specs=pl.BlockSpec((RB, 1), lambda i: (i, 0)),
        out_shape=jax.ShapeDtypeStruct((NN, 1), jnp.float32),
    )(out_raw, h_res, cb, g, be, w1, b1, w2, b2)


def kernel(params, weights, edge_index):
    w = weights
    loops = jnp.arange(NN, dtype=jnp.int32)
    src = jnp.concatenate([edge_index[0].astype(jnp.int32), loops])
    dst = jnp.concatenate([edge_index[1].astype(jnp.int32), loops])
    key = jnp.sort((dst << 15) | src)
    src_s = key & 0x7FFF
    dst_s = key >> 15
    wb = jnp.searchsorted(dst_s, jnp.arange(0, NN + 1, NPW, dtype=jnp.int32)
                          ).astype(jnp.int32)
    src_sp = jnp.concatenate([src_s, jnp.zeros((K,), jnp.int32)])
    dpack = dst_s | ((dst_s % (NPW // 2)) * 128 << 15)
    dst_sp = jnp.concatenate([dpack, jnp.zeros((K,), jnp.int32)])
    wb_pad = jnp.concatenate([wb, jnp.full((23,), ET, jnp.int32)])

    p4 = jnp.pad(params, ((0, 0), (0, 1)))
    w1 = jnp.pad(w['enc_W1'], ((0, 1), (0, 0)))
    hflat = _enc_call(p4, {**w, 'enc_W1': w1,
                           'enc_b1': w['enc_b1'][None, :],
                           'enc_b2': w['enc_b2'][None, :],
                           'enc_b3': w['enc_b3'][None, :]})
    h = hflat.reshape(NN, HID)

    xl, xr = _proj_call(h, w['Wl0'], w['bl0'][None, :],
                        w['Wr0'], w['br0'][None, :])
    for l in range(NLAYERS):
        att_flat = w[f'att{l}'].reshape(128)
        out0, out1 = _make_edge_call()(xl, xr.reshape(NN * 128), src_sp,
                                       dst_sp, wb_pad, att_flat)
        out_raw = jnp.concatenate([out0, out1]).reshape(NN, HID)
        if l < NLAYERS - 1:
            h, xl, xr = _postproj_call(
                out_raw, h, w[f'cb{l}'][None, :], w[f'g{l}'][None, :],
                w[f'be{l}'][None, :], w[f'Wl{l+1}'], w[f'bl{l+1}'][None, :],
                w[f'Wr{l+1}'], w[f'br{l+1}'][None, :])
        else:
            pred = _postdec_call(
                out_raw, h, w[f'cb{l}'][None, :], w[f'g{l}'][None, :],
                w[f'be{l}'][None, :], w['dec_W1'], w['dec_b1'][None, :],
                w['dec_W2'], w['dec_b2'][None, :])
    return pred.reshape(NN)
